# baseline (speedup 1.0000x reference)
import jax
import jax.numpy as jnp
from jax import lax
from jax.experimental import pallas as pl
from jax.experimental.pallas import tpu as pltpu
from jax.experimental.pallas import tpu_sc as plsc

N = 10000
E = 320000
C_IN = 128
C_OUT = 128
K_VOL = 27
EPS = 1e-5

_CH = 128
_NW = 32
_SEG = 2
_NCH = 2 * _SEG * (-(-E // (_NW * _CH * 2 * _SEG)))
_CPS = _NCH // _SEG
_EPT = _NCH * _CH
_E_PAD = _EPT * _NW
_ACC_ROWS = 10240
_ROWS_PER_TILE = _ACC_ROWS // 16


def _xw_body(x_ref, w_ref, o_ref):
    o_ref[0] = jnp.dot(x_ref[...], w_ref[0], preferred_element_type=jnp.float32)


def _bn_body(p_ref, g_ref, b_ref, o_ref):
    s = p_ref[0, :N] + p_ref[1, :N]
    mean = jnp.mean(s, axis=0, keepdims=True)
    d = s - mean
    var = jnp.mean(d * d, axis=0, keepdims=True)
    o_ref[...] = d / jnp.sqrt(var + EPS) * g_ref[...] + b_ref[...]


def _sc_body(xw_hbm, idx4_hbm, dst4_hbm, out_hbm,
             idx2_v, dst2_v, rows0_v, rows1_v, acc_sh, sem0, sem1):
    cid = lax.axis_index("c")
    sid = lax.axis_index("s")
    wid = sid * 2 + cid

    def zrow(r, c):
        for j in range(C_OUT // 16):
            rows0_v[r, pl.ds(j * 16, 16)] = jnp.zeros((16,), jnp.float32)
        return c
    lax.fori_loop(0, _CH, zrow, 0)
    r0 = sid * _ROWS_PER_TILE
    for z in range(_ROWS_PER_TILE // _CH):
        pltpu.sync_copy(rows0_v, acc_sh.at[pl.ds(r0 + z * _CH, _CH)])
    plsc.subcore_barrier()

    for seg in range(_SEG):
        pltpu.sync_copy(idx4_hbm.at[wid, seg], idx2_v)
        pltpu.sync_copy(dst4_hbm.at[wid, seg], dst2_v)
        pltpu.async_copy(xw_hbm.at[idx2_v.at[0]], rows0_v, sem0)

        def pipe(h, c):
            g = h * 2
            pltpu.async_copy(xw_hbm.at[idx2_v.at[g + 1]], rows1_v, sem1)
            pltpu.make_async_copy(xw_hbm.at[idx2_v.at[g]], rows0_v, sem0).wait()
            pltpu.sync_copy(rows0_v, acc_sh.at[dst2_v.at[g]], add=True)

            @pl.when(g + 2 < _CPS)
            def _():
                pltpu.async_copy(xw_hbm.at[idx2_v.at[g + 2]], rows0_v, sem0)

            pltpu.make_async_copy(xw_hbm.at[idx2_v.at[g + 1]], rows1_v, sem1).wait()
            pltpu.sync_copy(rows1_v, acc_sh.at[dst2_v.at[g + 1]], add=True)
            return c
        lax.fori_loop(0, _CPS // 2, pipe, 0)

    plsc.subcore_barrier()
    pltpu.sync_copy(acc_sh.at[pl.ds(r0, _ROWS_PER_TILE)],
                    out_hbm.at[cid, pl.ds(r0, _ROWS_PER_TILE)])


def kernel(x, W, gamma, beta, edge_index, kernel_offsets):
    ppt = _EPT - E // _NW
    idx2 = (kernel_offsets * N + edge_index[0]).reshape(_NW, E // _NW)
    dst2 = edge_index[1].reshape(_NW, E // _NW)
    pad_idx = ((jnp.arange(_NW, dtype=jnp.int32)[:, None] * ppt
                + jnp.arange(ppt, dtype=jnp.int32)[None, :]) % (K_VOL * N))
    pad_dst = jnp.broadcast_to(
        N + jnp.arange(ppt, dtype=jnp.int32) % (_ACC_ROWS - N), (_NW, ppt))
    idx4 = jnp.concatenate([idx2, pad_idx], axis=1).reshape(_NW, _SEG, _CPS, _CH)
    dst4 = jnp.concatenate([dst2, pad_dst], axis=1).reshape(_NW, _SEG, _CPS, _CH)

    block_n = 2000
    xw = pl.pallas_call(
        _xw_body,
        grid=(N // block_n, K_VOL),
        in_specs=[pl.BlockSpec((block_n, C_IN), lambda nb, k: (nb, 0)),
                  pl.BlockSpec((1, C_IN, C_OUT), lambda nb, k: (k, 0, 0))],
        out_specs=pl.BlockSpec((1, block_n, C_OUT), lambda nb, k: (k, nb, 0)),
        out_shape=jax.ShapeDtypeStruct((K_VOL, N, C_OUT), jnp.float32),
    )(x.astype(jnp.bfloat16), W.astype(jnp.bfloat16))
    xw_flat = xw.reshape(K_VOL * N, C_OUT)

    mesh = plsc.VectorSubcoreMesh(core_axis_name="c", subcore_axis_name="s")
    partial = pl.kernel(
        _sc_body,
        out_type=jax.ShapeDtypeStruct((2, _ACC_ROWS, C_OUT), jnp.float32),
        mesh=mesh,
        scratch_types=[
            pltpu.VMEM((_CPS, _CH), jnp.int32),
            pltpu.VMEM((_CPS, _CH), jnp.int32),
            pltpu.VMEM((_CH, C_OUT), jnp.float32),
            pltpu.VMEM((_CH, C_OUT), jnp.float32),
            pltpu.VMEM_SHARED((_ACC_ROWS, C_OUT), jnp.float32),
            pltpu.SemaphoreType.DMA,
            pltpu.SemaphoreType.DMA,
        ],
    )(xw_flat, idx4, dst4)

    return pl.pallas_call(
        _bn_body,
        out_shape=jax.ShapeDtypeStruct((N, C_OUT), jnp.float32),
    )(partial, gamma.reshape(1, C_OUT), beta.reshape(1, C_OUT))


# speedup vs baseline: 1.2950x; 1.2950x over previous
import jax
import jax.numpy as jnp
from jax import lax
from jax.experimental import pallas as pl
from jax.experimental.pallas import tpu as pltpu
from jax.experimental.pallas import tpu_sc as plsc

N = 10000
E = 320000
C_IN = 128
C_OUT = 128
K_VOL = 27
EPS = 1e-5

_CH = 128
_NW = 32
_SEG = 2
_NCH = 2 * _SEG * (-(-E // (_NW * _CH * 2 * _SEG)))
_CPS = _NCH // _SEG
_EPT = _NCH * _CH
_E_PAD = _EPT * _NW
_ACC_ROWS = 10240
_ROWS_PER_TILE = _ACC_ROWS // 16


def _xw_body(x_ref, w_ref, o_ref):
    for k in range(K_VOL):
        o_ref[k] = jnp.dot(x_ref[...], w_ref[k],
                           preferred_element_type=jnp.float32)


def _bn_body(p_ref, g_ref, b_ref, o_ref):
    s = p_ref[0, :N] + p_ref[1, :N]
    mean = jnp.mean(s, axis=0, keepdims=True)
    d = s - mean
    var = jnp.mean(d * d, axis=0, keepdims=True)
    o_ref[...] = d / jnp.sqrt(var + EPS) * g_ref[...] + b_ref[...]


def _sc_body(xw_hbm, idx4_hbm, dst4_hbm, out_hbm,
             idx2_v, dst2_v, rows0_v, rows1_v, acc_sh, sem0, sem1):
    cid = lax.axis_index("c")
    sid = lax.axis_index("s")
    wid = sid * 2 + cid

    def zrow(r, c):
        for j in range(C_OUT // 16):
            rows0_v[r, pl.ds(j * 16, 16)] = jnp.zeros((16,), jnp.float32)
        return c
    lax.fori_loop(0, _CH, zrow, 0)
    r0 = sid * _ROWS_PER_TILE
    for z in range(_ROWS_PER_TILE // _CH):
        pltpu.sync_copy(rows0_v, acc_sh.at[pl.ds(r0 + z * _CH, _CH)])
    plsc.subcore_barrier()

    for seg in range(_SEG):
        pltpu.sync_copy(idx4_hbm.at[wid, seg], idx2_v)
        pltpu.sync_copy(dst4_hbm.at[wid, seg], dst2_v)
        pltpu.async_copy(xw_hbm.at[idx2_v.at[0]], rows0_v, sem0)

        def pipe(h, c):
            g = h * 2
            pltpu.async_copy(xw_hbm.at[idx2_v.at[g + 1]], rows1_v, sem1)
            pltpu.make_async_copy(xw_hbm.at[idx2_v.at[g]], rows0_v, sem0).wait()
            pltpu.sync_copy(rows0_v, acc_sh.at[dst2_v.at[g]], add=True)

            @pl.when(g + 2 < _CPS)
            def _():
                pltpu.async_copy(xw_hbm.at[idx2_v.at[g + 2]], rows0_v, sem0)

            pltpu.make_async_copy(xw_hbm.at[idx2_v.at[g + 1]], rows1_v, sem1).wait()
            pltpu.sync_copy(rows1_v, acc_sh.at[dst2_v.at[g + 1]], add=True)
            return c
        lax.fori_loop(0, _CPS // 2, pipe, 0)

    plsc.subcore_barrier()
    pltpu.sync_copy(acc_sh.at[pl.ds(r0, _ROWS_PER_TILE)],
                    out_hbm.at[cid, pl.ds(r0, _ROWS_PER_TILE)])


def kernel(x, W, gamma, beta, edge_index, kernel_offsets):
    ppt = _EPT - E // _NW
    idx2 = (kernel_offsets * N + edge_index[0]).reshape(_NW, E // _NW)
    dst2 = edge_index[1].reshape(_NW, E // _NW)
    pad_idx = ((jnp.arange(_NW, dtype=jnp.int32)[:, None] * ppt
                + jnp.arange(ppt, dtype=jnp.int32)[None, :]) % (K_VOL * N))
    pad_dst = jnp.broadcast_to(
        N + jnp.arange(ppt, dtype=jnp.int32) % (_ACC_ROWS - N), (_NW, ppt))
    idx4 = jnp.concatenate([idx2, pad_idx], axis=1).reshape(_NW, _SEG, _CPS, _CH)
    dst4 = jnp.concatenate([dst2, pad_dst], axis=1).reshape(_NW, _SEG, _CPS, _CH)

    block_n = 1000
    xw = pl.pallas_call(
        _xw_body,
        grid=(N // block_n,),
        in_specs=[pl.BlockSpec((block_n, C_IN), lambda nb: (nb, 0)),
                  pl.BlockSpec((K_VOL, C_IN, C_OUT), lambda nb: (0, 0, 0))],
        out_specs=pl.BlockSpec((K_VOL, block_n, C_OUT), lambda nb: (0, nb, 0)),
        out_shape=jax.ShapeDtypeStruct((K_VOL, N, C_OUT), jnp.float32),
    )(x.astype(jnp.bfloat16), W.astype(jnp.bfloat16))
    xw_flat = xw.reshape(K_VOL * N, C_OUT)

    mesh = plsc.VectorSubcoreMesh(core_axis_name="c", subcore_axis_name="s")
    partial = pl.kernel(
        _sc_body,
        out_type=jax.ShapeDtypeStruct((2, _ACC_ROWS, C_OUT), jnp.float32),
        mesh=mesh,
        scratch_types=[
            pltpu.VMEM((_CPS, _CH), jnp.int32),
            pltpu.VMEM((_CPS, _CH), jnp.int32),
            pltpu.VMEM((_CH, C_OUT), jnp.float32),
            pltpu.VMEM((_CH, C_OUT), jnp.float32),
            pltpu.VMEM_SHARED((_ACC_ROWS, C_OUT), jnp.float32),
            pltpu.SemaphoreType.DMA,
            pltpu.SemaphoreType.DMA,
        ],
    )(xw_flat, idx4, dst4)

    return pl.pallas_call(
        _bn_body,
        out_shape=jax.ShapeDtypeStruct((N, C_OUT), jnp.float32),
    )(partial, gamma.reshape(1, C_OUT), beta.reshape(1, C_OUT))


# speedup vs baseline: 1.3276x; 1.0252x over previous
import numpy as np
import jax
import jax.numpy as jnp
from jax import lax
from jax.experimental import pallas as pl
from jax.experimental.pallas import tpu as pltpu
from jax.experimental.pallas import tpu_sc as plsc

N = 10000
E = 320000
C_IN = 128
C_OUT = 128
K_VOL = 27
EPS = 1e-5

_CH = 128
_NW = 32
_SEG = 2
_REAL_PT = E // _NW
_SEG_E = 5120
_CPS = _SEG_E // _CH
_PAD_PT = _SEG * _SEG_E - _REAL_PT
_REM = _REAL_PT - _SEG_E
_ACC_ROWS = 10240
_ROWS_PER_TILE = _ACC_ROWS // 16

_PAD_IDX = np.arange(_NW * _PAD_PT, dtype=np.int32) % (K_VOL * N)
_PAD_DST = np.tile(N + np.arange(_PAD_PT, dtype=np.int32), _NW)


def _xw_body(x_ref, w_ref, o_ref):
    xb = x_ref[...].astype(jnp.bfloat16)
    for k in range(K_VOL):
        o_ref[k] = jnp.dot(xb, w_ref[k].astype(jnp.bfloat16),
                           preferred_element_type=jnp.float32)


def _bn_body(p_ref, g_ref, b_ref, o_ref):
    s = p_ref[0, :N] + p_ref[1, :N]
    mean = jnp.mean(s, axis=0, keepdims=True)
    d = s - mean
    var = jnp.mean(d * d, axis=0, keepdims=True)
    o_ref[...] = d / jnp.sqrt(var + EPS) * g_ref[...] + b_ref[...]


def _sc_body(xw_hbm, idx_hbm, dst_hbm, pidx_hbm, pdst_hbm, out_hbm,
             idx_v, dst_v, dstb0_v, dstb1_v, rows0_v, rows1_v,
             acc_sh, sem0, sem1):
    cid = lax.axis_index("c")
    sid = lax.axis_index("s")
    wid = sid * 2 + cid

    def zrow(r, c):
        for j in range(C_OUT // 16):
            rows0_v[r, pl.ds(j * 16, 16)] = jnp.zeros((16,), jnp.float32)
        return c
    lax.fori_loop(0, _CH, zrow, 0)
    r0 = sid * _ROWS_PER_TILE
    for z in range(_ROWS_PER_TILE // _CH):
        pltpu.sync_copy(rows0_v, acc_sh.at[pl.ds(r0 + z * _CH, _CH)])
    plsc.subcore_barrier()

    for seg in range(_SEG):
        if seg == 0:
            pltpu.sync_copy(idx_hbm.at[pl.ds(wid * _REAL_PT, _SEG_E)], idx_v)
            pltpu.sync_copy(dst_hbm.at[pl.ds(wid * _REAL_PT, _SEG_E)], dst_v)
        else:
            b = wid * _REAL_PT + _SEG_E
            pltpu.sync_copy(idx_hbm.at[pl.ds(b, _REM)], idx_v.at[pl.ds(0, _REM)])
            pltpu.sync_copy(dst_hbm.at[pl.ds(b, _REM)], dst_v.at[pl.ds(0, _REM)])
            pltpu.sync_copy(pidx_hbm.at[pl.ds(wid * _PAD_PT, _PAD_PT)],
                            idx_v.at[pl.ds(_REM, _PAD_PT)])
            pltpu.sync_copy(pdst_hbm.at[pl.ds(wid * _PAD_PT, _PAD_PT)],
                            dst_v.at[pl.ds(_REM, _PAD_PT)])

        pltpu.async_copy(xw_hbm.at[idx_v.at[pl.ds(0, _CH)]], rows0_v, sem0)

        def pipe(h, c):
            g = h * 2
            pltpu.async_copy(
                xw_hbm.at[idx_v.at[pl.ds((g + 1) * _CH, _CH)]], rows1_v, sem1)
            for j in range(_CH // 16):
                dstb0_v[pl.ds(j * 16, 16)] = dst_v[pl.ds(g * _CH + j * 16, 16)]
            pltpu.make_async_copy(
                xw_hbm.at[idx_v.at[pl.ds(g * _CH, _CH)]], rows0_v, sem0).wait()
            pltpu.sync_copy(rows0_v, acc_sh.at[dstb0_v], add=True)

            @pl.when(g + 2 < _CPS)
            def _():
                pltpu.async_copy(
                    xw_hbm.at[idx_v.at[pl.ds((g + 2) * _CH, _CH)]], rows0_v, sem0)

            for j in range(_CH // 16):
                dstb1_v[pl.ds(j * 16, 16)] = dst_v[pl.ds((g + 1) * _CH + j * 16, 16)]
            pltpu.make_async_copy(
                xw_hbm.at[idx_v.at[pl.ds((g + 1) * _CH, _CH)]], rows1_v, sem1).wait()
            pltpu.sync_copy(rows1_v, acc_sh.at[dstb1_v], add=True)
            return c
        lax.fori_loop(0, _CPS // 2, pipe, 0)

    plsc.subcore_barrier()
    pltpu.sync_copy(acc_sh.at[pl.ds(r0, _ROWS_PER_TILE)],
                    out_hbm.at[cid, pl.ds(r0, _ROWS_PER_TILE)])


def kernel(x, W, gamma, beta, edge_index, kernel_offsets):
    idx_flat = kernel_offsets * N + edge_index[0]
    dst_flat = edge_index[1]

    block_n = 1000
    xw = pl.pallas_call(
        _xw_body,
        grid=(N // block_n,),
        in_specs=[pl.BlockSpec((block_n, C_IN), lambda nb: (nb, 0)),
                  pl.BlockSpec((K_VOL, C_IN, C_OUT), lambda nb: (0, 0, 0))],
        out_specs=pl.BlockSpec((K_VOL, block_n, C_OUT), lambda nb: (0, nb, 0)),
        out_shape=jax.ShapeDtypeStruct((K_VOL, N, C_OUT), jnp.float32),
    )(x, W)
    xw_flat = xw.reshape(K_VOL * N, C_OUT)

    mesh = plsc.VectorSubcoreMesh(core_axis_name="c", subcore_axis_name="s")
    partial = pl.kernel(
        _sc_body,
        out_type=jax.ShapeDtypeStruct((2, _ACC_ROWS, C_OUT), jnp.float32),
        mesh=mesh,
        scratch_types=[
            pltpu.VMEM((_SEG_E,), jnp.int32),
            pltpu.VMEM((_SEG_E,), jnp.int32),
            pltpu.VMEM((_CH,), jnp.int32),
            pltpu.VMEM((_CH,), jnp.int32),
            pltpu.VMEM((_CH, C_OUT), jnp.float32),
            pltpu.VMEM((_CH, C_OUT), jnp.float32),
            pltpu.VMEM_SHARED((_ACC_ROWS, C_OUT), jnp.float32),
            pltpu.SemaphoreType.DMA,
            pltpu.SemaphoreType.DMA,
        ],
    )(xw_flat, idx_flat, dst_flat, jnp.asarray(_PAD_IDX), jnp.asarray(_PAD_DST))

    return pl.pallas_call(
        _bn_body,
        out_shape=jax.ShapeDtypeStruct((N, C_OUT), jnp.float32),
    )(partial, gamma.reshape(1, C_OUT), beta.reshape(1, C_OUT))


# speedup vs baseline: 1.4314x; 1.0782x over previous
import numpy as np
import jax
import jax.numpy as jnp
from jax import lax
from jax.experimental import pallas as pl
from jax.experimental.pallas import tpu as pltpu
from jax.experimental.pallas import tpu_sc as plsc

N = 10000
E = 320000
C_IN = 128
C_OUT = 128
K_VOL = 27
EPS = 1e-5

_CH = 128
_NW = 32
_SEG = 2
_REAL_PT = E // _NW
_SEG_E = 5120
_CPS = _SEG_E // _CH
_PAD_PT = _SEG * _SEG_E - _REAL_PT
_REM = _REAL_PT - _SEG_E
_ACC_ROWS = 10240
_ROWS_PER_TILE = _ACC_ROWS // 16

_PAD_IDX = np.arange(_NW * _PAD_PT, dtype=np.int32) % (K_VOL * N)
_PAD_DST = np.tile(N + np.arange(_PAD_PT, dtype=np.int32), _NW)


def _xw_body(x_ref, w_ref, e_ref, off_ref, o_ref, idx_ref, dst_ref):
    xb = x_ref[...].astype(jnp.bfloat16)
    for k in range(K_VOL):
        o_ref[k] = jnp.dot(xb, w_ref[k].astype(jnp.bfloat16),
                           preferred_element_type=jnp.float32)
    @pl.when(pl.program_id(0) == 0)
    def _():
        idx_ref[...] = off_ref[...] * N + e_ref[0]
        dst_ref[...] = e_ref[1]


def _bn_body(p_ref, g_ref, b_ref, o_ref):
    s = p_ref[0, :N] + p_ref[1, :N]
    mean = jnp.mean(s, axis=0, keepdims=True)
    d = s - mean
    var = jnp.mean(d * d, axis=0, keepdims=True)
    o_ref[...] = d / jnp.sqrt(var + EPS) * g_ref[...] + b_ref[...]


def _sc_body(xw_hbm, idx_hbm, dst_hbm, pidx_hbm, pdst_hbm, out_hbm,
             idx_v, dst_v, dstb0_v, dstb1_v, rows0_v, rows1_v,
             acc_sh, sem0, sem1):
    cid = lax.axis_index("c")
    sid = lax.axis_index("s")
    wid = sid * 2 + cid

    def zrow(r, c):
        for j in range(C_OUT // 16):
            rows0_v[r, pl.ds(j * 16, 16)] = jnp.zeros((16,), jnp.float32)
        return c
    lax.fori_loop(0, _CH, zrow, 0)
    r0 = sid * _ROWS_PER_TILE
    for z in range(_ROWS_PER_TILE // _CH):
        pltpu.sync_copy(rows0_v, acc_sh.at[pl.ds(r0 + z * _CH, _CH)])
    plsc.subcore_barrier()

    for seg in range(_SEG):
        if seg == 0:
            pltpu.sync_copy(idx_hbm.at[pl.ds(wid * _REAL_PT, _SEG_E)], idx_v)
            pltpu.sync_copy(dst_hbm.at[pl.ds(wid * _REAL_PT, _SEG_E)], dst_v)
        else:
            b = wid * _REAL_PT + _SEG_E
            pltpu.sync_copy(idx_hbm.at[pl.ds(b, _REM)], idx_v.at[pl.ds(0, _REM)])
            pltpu.sync_copy(dst_hbm.at[pl.ds(b, _REM)], dst_v.at[pl.ds(0, _REM)])
            pltpu.sync_copy(pidx_hbm.at[pl.ds(wid * _PAD_PT, _PAD_PT)],
                            idx_v.at[pl.ds(_REM, _PAD_PT)])
            pltpu.sync_copy(pdst_hbm.at[pl.ds(wid * _PAD_PT, _PAD_PT)],
                            dst_v.at[pl.ds(_REM, _PAD_PT)])

        pltpu.async_copy(xw_hbm.at[idx_v.at[pl.ds(0, _CH)]], rows0_v, sem0)

        def pipe(h, c):
            g = h * 2
            pltpu.async_copy(
                xw_hbm.at[idx_v.at[pl.ds((g + 1) * _CH, _CH)]], rows1_v, sem1)
            for j in range(_CH // 16):
                dstb0_v[pl.ds(j * 16, 16)] = dst_v[pl.ds(g * _CH + j * 16, 16)]
            pltpu.make_async_copy(
                xw_hbm.at[idx_v.at[pl.ds(g * _CH, _CH)]], rows0_v, sem0).wait()
            pltpu.sync_copy(rows0_v, acc_sh.at[dstb0_v], add=True)

            @pl.when(g + 2 < _CPS)
            def _():
                pltpu.async_copy(
                    xw_hbm.at[idx_v.at[pl.ds((g + 2) * _CH, _CH)]], rows0_v, sem0)

            for j in range(_CH // 16):
                dstb1_v[pl.ds(j * 16, 16)] = dst_v[pl.ds((g + 1) * _CH + j * 16, 16)]
            pltpu.make_async_copy(
                xw_hbm.at[idx_v.at[pl.ds((g + 1) * _CH, _CH)]], rows1_v, sem1).wait()
            pltpu.sync_copy(rows1_v, acc_sh.at[dstb1_v], add=True)
            return c
        lax.fori_loop(0, _CPS // 2, pipe, 0)

    plsc.subcore_barrier()
    pltpu.sync_copy(acc_sh.at[pl.ds(r0, _ROWS_PER_TILE)],
                    out_hbm.at[cid, pl.ds(r0, _ROWS_PER_TILE)])


def kernel(x, W, gamma, beta, edge_index, kernel_offsets):
    block_n = 1000
    block_e = E // (N // block_n)
    xw, idx_flat, dst_flat = pl.pallas_call(
        _xw_body,
        grid=(N // block_n,),
        in_specs=[pl.BlockSpec((block_n, C_IN), lambda nb: (nb, 0)),
                  pl.BlockSpec((K_VOL, C_IN, C_OUT), lambda nb: (0, 0, 0)),
                  pl.BlockSpec((2, E), lambda nb: (0, 0)),
                  pl.BlockSpec((E,), lambda nb: (0,))],
        out_specs=[pl.BlockSpec((K_VOL, block_n, C_OUT), lambda nb: (0, nb, 0)),
                   pl.BlockSpec((E,), lambda nb: (0,)),
                   pl.BlockSpec((E,), lambda nb: (0,))],
        out_shape=[jax.ShapeDtypeStruct((K_VOL, N, C_OUT), jnp.float32),
                   jax.ShapeDtypeStruct((E,), jnp.int32),
                   jax.ShapeDtypeStruct((E,), jnp.int32)],
    )(x, W, edge_index, kernel_offsets)
    xw_flat = xw.reshape(K_VOL * N, C_OUT)

    mesh = plsc.VectorSubcoreMesh(core_axis_name="c", subcore_axis_name="s")
    partial = pl.kernel(
        _sc_body,
        out_type=jax.ShapeDtypeStruct((2, _ACC_ROWS, C_OUT), jnp.float32),
        mesh=mesh,
        scratch_types=[
            pltpu.VMEM((_SEG_E,), jnp.int32),
            pltpu.VMEM((_SEG_E,), jnp.int32),
            pltpu.VMEM((_CH,), jnp.int32),
            pltpu.VMEM((_CH,), jnp.int32),
            pltpu.VMEM((_CH, C_OUT), jnp.float32),
            pltpu.VMEM((_CH, C_OUT), jnp.float32),
            pltpu.VMEM_SHARED((_ACC_ROWS, C_OUT), jnp.float32),
            pltpu.SemaphoreType.DMA,
            pltpu.SemaphoreType.DMA,
        ],
    )(xw_flat, idx_flat, dst_flat, jnp.asarray(_PAD_IDX), jnp.asarray(_PAD_DST))

    return pl.pallas_call(
        _bn_body,
        out_shape=jax.ShapeDtypeStruct((N, C_OUT), jnp.float32),
    )(partial, gamma.reshape(1, C_OUT), beta.reshape(1, C_OUT))
